# baseline (speedup 1.0000x reference)
import functools

import jax
import jax.numpy as jnp
from jax import lax
from jax.experimental import pallas as pl
from jax.experimental.pallas import tpu as pltpu
from jax.experimental.pallas import tpu_sc as plsc

HEADS = 16
BUCKET_SIZE = 64
N_HASHES = 4
TOKEN_SELF_ATTN_VALUE = -5e4

B, N, C = 2, 2048, 1024
D = C // HEADS
BH = B * HEADS
NB = N // BUCKET_SIZE
NCH = N_HASHES * NB
ROWS = BH * N_HASHES
NW = 32
RPW = ROWS // NW
GCH = 128
NG = N // GCH


def _proj_body(q_ref, wqk_ref, wv_ref, rot_ref, qkv_ref, bkt_ref):
    x = q_ref[0]
    qk = jnp.dot(x, wqk_ref[...], preferred_element_type=jnp.float32)
    v = jnp.dot(x, wv_ref[...], preferred_element_type=jnp.float32)
    rot = rot_ref[...]
    c = lax.broadcasted_iota(jnp.int32, (128, 4), 0)
    hcol = lax.broadcasted_iota(jnp.int32, (128, 4), 1)
    pw = lax.bitcast_convert_type((158 - c // 4) << 23, jnp.float32)
    selw = jnp.where((c % 4) == hcol, pw, 0.0)
    for h in range(HEADS):
        qh = qk[:, h * D:(h + 1) * D]
        qkv_ref[h] = jnp.concatenate([qh, v[:, h * D:(h + 1) * D]], axis=1)
        xf = jnp.dot(qh, rot, preferred_element_type=jnp.float32)
        t = jnp.maximum(xf[:, 0:64], xf[:, 64:128])
        t = jnp.maximum(t[:, 0:32], t[:, 32:64])
        t = jnp.maximum(t[:, 0:16], t[:, 16:32])
        t = jnp.maximum(t[:, 0:8], t[:, 8:16])
        mb = jnp.maximum(t[:, 0:4], t[:, 4:8])
        for _ in range(5):
            mb = jnp.concatenate([mb, mb], axis=1)
        eq = jnp.where(xf == mb, 1.0, 0.0)
        sc_ = jnp.dot(eq, selw, preferred_element_type=jnp.float32)
        scT = jnp.transpose(sc_, (1, 0))
        bkt_ref[h] = 158 - (lax.bitcast_convert_type(scT, jnp.int32) >> 23)


def _stage1(queries, Wqk, Wv, rot2):
    tiles = N // 256
    grid = (B * tiles,)
    return pl.pallas_call(
        _proj_body,
        grid=grid,
        in_specs=[
            pl.BlockSpec((1, 256, C), lambda i: (i // tiles, i % tiles, 0)),
            pl.BlockSpec((C, C), lambda i: (0, 0)),
            pl.BlockSpec((C, C), lambda i: (0, 0)),
            pl.BlockSpec((D, 128), lambda i: (0, 0)),
        ],
        out_specs=[
            pl.BlockSpec((HEADS, 256, 2 * D), lambda i: (i // tiles, i % tiles, 0)),
            pl.BlockSpec((HEADS, N_HASHES, 256), lambda i: (i // tiles, 0, i % tiles)),
        ],
        out_shape=[
            jax.ShapeDtypeStruct((BH, N, 2 * D), jnp.float32),
            jax.ShapeDtypeStruct((BH, N_HASHES, N), jnp.int32),
        ],
    )(queries, Wqk, Wv, rot2)


def _sort_gather_body(bkt_hbm, qkv_hbm,
                      sti_hbm, stf_hbm, sqkv_hbm,
                      bktv, stv, stfv, gidxv, cntv, curv, gbuf, sem):
    cc = lax.axis_index("c")
    ss = lax.axis_index("s")
    wid = ss * 2 + cc
    zeros16 = jnp.zeros((16,), jnp.int32)
    for rr in range(RPW):
        row = wid * RPW + rr
        bh = row // N_HASHES
        base = row * N
        pltpu.sync_copy(bkt_hbm.at[pl.ds(base, N)], bktv)
        cntv[pl.ds(0, 16)] = zeros16
        cntv[pl.ds(16, 16)] = zeros16

        def p1(t, carry):
            vec = bktv[pl.ds(t * 16, 16)]
            sc, lastm = plsc.scan_count(vec)
            plsc.addupdate_scatter(cntv, [vec], sc, mask=lastm)
            return carry

        lax.fori_loop(0, N // 16, p1, 0)

        c0 = cntv[pl.ds(0, 16)]
        c1 = cntv[pl.ds(16, 16)]
        ic0 = plsc.cumsum(c0)
        ic1 = plsc.cumsum(c1)
        tot0 = jnp.sum(c0)
        curv[pl.ds(0, 16)] = ic0 - c0
        curv[pl.ds(16, 16)] = ic1 - c1 + tot0

        gbase = bh * N

        def p2(t, carry):
            vec = bktv[pl.ds(t * 16, 16)]
            sc, lastm = plsc.scan_count(vec)
            bse = plsc.load_gather(curv, [vec])
            dest = bse + sc - 1
            tok = t * 16 + lax.iota(jnp.int32, 16)
            plsc.store_scatter(stv, [dest], tok)
            plsc.store_scatter(stfv, [dest], tok.astype(jnp.float32))
            plsc.store_scatter(gidxv, [dest], tok + gbase)
            plsc.store_scatter(curv, [vec], bse + sc, mask=lastm)
            return carry

        lax.fori_loop(0, N // 16, p2, 0)

        pltpu.sync_copy(stv, sti_hbm.at[pl.ds(base, N)])
        pltpu.sync_copy(stfv, stf_hbm.at[pl.ds(base, N)])
        for k in range(NG):
            idx = gidxv.at[pl.ds(k * GCH, GCH)]
            pltpu.async_copy(qkv_hbm.at[idx], gbuf, sem).wait()
            pltpu.sync_copy(gbuf, sqkv_hbm.at[pl.ds(base + k * GCH, GCH)])


def _stage2(bkt_flat, qkv_flat):
    mesh = plsc.VectorSubcoreMesh(core_axis_name="c", subcore_axis_name="s")
    f = functools.partial(
        pl.kernel,
        mesh=mesh,
        out_type=[
            jax.ShapeDtypeStruct((ROWS * N,), jnp.int32),
            jax.ShapeDtypeStruct((ROWS * N,), jnp.float32),
            jax.ShapeDtypeStruct((ROWS * N, 2 * D), jnp.float32),
        ],
        scratch_types=[
            pltpu.VMEM((N,), jnp.int32),
            pltpu.VMEM((N,), jnp.int32),
            pltpu.VMEM((N,), jnp.float32),
            pltpu.VMEM((N,), jnp.int32),
            pltpu.VMEM((NB,), jnp.int32),
            pltpu.VMEM((NB,), jnp.int32),
            pltpu.VMEM((GCH, 2 * D), jnp.float32),
            pltpu.SemaphoreType.DMA,
        ],
        compiler_params=pltpu.CompilerParams(needs_layout_passes=False),
    )(_sort_gather_body)
    return f(bkt_flat, qkv_flat)


ATT_G = 2


def _attn_body(sqkv_ref, stf_ref, sow_ref):
    G = ATT_G
    M = G * BUCKET_SIZE
    W = (G + 1) * BUCKET_SIZE
    ii = lax.broadcasted_iota(jnp.int32, (M, W), 0)
    jj = lax.broadcasted_iota(jnp.int32, (M, W), 1)
    qc = ii // BUCKET_SIZE
    kc = jj // BUCKET_SIZE
    band = jnp.logical_or(kc == qc, kc == qc + 1)
    selfd = jj == ii + BUCKET_SIZE
    repl_mask = jnp.logical_or(selfd, jnp.logical_not(band))
    repl_val = jnp.where(selfd, TOKEN_SELF_ATTN_VALUE, -1e30)
    for g in range(NCH // G):
        c0 = g * G
        pc = (c0 - 1) % NCH
        qall = sqkv_ref[0, c0 * BUCKET_SIZE:c0 * BUCKET_SIZE + M, :]
        q = qall[:, 0:D]
        vcur = qall[:, D:2 * D]
        pall = sqkv_ref[0, pc * BUCKET_SIZE:(pc + 1) * BUCKET_SIZE, :]
        kcat = jnp.concatenate([pall[:, 0:D], q], axis=0)
        nrm = jnp.sqrt(jnp.sum(kcat * kcat, axis=1, keepdims=True))
        kn = kcat / (jnp.maximum(nrm, 1e-12) * (D ** 0.5))
        dots = lax.dot_general(
            q.astype(jnp.bfloat16), kn.astype(jnp.bfloat16),
            (((1,), (1,)), ((), ())),
            preferred_element_type=jnp.float32)
        dots = jnp.where(repl_mask, repl_val, dots)
        if c0 % NB == 0:
            qid = jnp.reshape(stf_ref[0, pl.ds(c0, 1), :],
                              (BUCKET_SIZE, 1))
            pid = stf_ref[0, pl.ds(pc, 1), :]
            eq = jnp.where(qid == pid, 1.0, 0.0)
            eq = jnp.concatenate(
                [eq, jnp.zeros((BUCKET_SIZE, W - BUCKET_SIZE), jnp.float32)],
                axis=1)
            eq = jnp.concatenate(
                [eq, jnp.zeros((M - BUCKET_SIZE, W), jnp.float32)], axis=0)
            dots = jnp.where(eq > 0.5, TOKEN_SELF_ATTN_VALUE, dots)
        m = jnp.max(dots, axis=1, keepdims=True)
        ex = jnp.exp(dots - m)
        ssum = jnp.sum(ex, axis=1, keepdims=True)
        lse = m + jnp.log(ssum)
        p = ex * (1.0 / ssum)
        vcat = jnp.concatenate([pall[:, D:2 * D], vcur], axis=0)
        bo = lax.dot_general(
            p.astype(jnp.bfloat16), vcat.astype(jnp.bfloat16),
            (((1,), (0,)), ((), ())),
            preferred_element_type=jnp.float32)
        packed = jnp.concatenate(
            [bo, jnp.broadcast_to(lse, (M, D))], axis=1)
        sow_ref[0, c0 * BUCKET_SIZE:c0 * BUCKET_SIZE + M, :] = packed


def _stage3(sqkv3, stf3):
    return pl.pallas_call(
        _attn_body,
        grid=(BH,),
        in_specs=[
            pl.BlockSpec((1, N_HASHES * N, 2 * D), lambda i: (i, 0, 0)),
            pl.BlockSpec((1, NCH, BUCKET_SIZE), lambda i: (i, 0, 0)),
        ],
        out_specs=[
            pl.BlockSpec((1, N_HASHES * N, 2 * D), lambda i: (i, 0, 0)),
        ],
        out_shape=[
            jax.ShapeDtypeStruct((BH, N_HASHES * N, 2 * D), jnp.float32),
        ],
    )(sqkv3, stf3)


def _unsort_body(sti_hbm, sow_hbm, ouw_hbm, stv, sidx, obuf, sem):
    cc = lax.axis_index("c")
    ss = lax.axis_index("s")
    wid = ss * 2 + cc
    for rr in range(RPW):
        row = wid * RPW + rr
        base = row * N
        pltpu.sync_copy(sti_hbm.at[pl.ds(base, N)], stv)

        for k in range(NG):
            def sk(j, carry):
                sidx[k, pl.ds(j * 16, 16)] = (
                    stv[pl.ds(k * GCH + j * 16, 16)] + base)
                return carry

            lax.fori_loop(0, GCH // 16, sk, 0)

        for k in range(NG):
            pltpu.sync_copy(sow_hbm.at[pl.ds(base + k * GCH, GCH)], obuf)
            pltpu.async_copy(obuf, ouw_hbm.at[sidx.at[k]], sem).wait()


def _stage4(sti_flat, sow_flat):
    mesh = plsc.VectorSubcoreMesh(core_axis_name="c", subcore_axis_name="s")
    f = functools.partial(
        pl.kernel,
        mesh=mesh,
        out_type=[
            jax.ShapeDtypeStruct((ROWS * N, 2 * D), jnp.float32),
        ],
        scratch_types=[
            pltpu.VMEM((N,), jnp.int32),
            pltpu.VMEM((NG, GCH), jnp.int32),
            pltpu.VMEM((GCH, 2 * D), jnp.float32),
            pltpu.SemaphoreType.DMA,
        ],
        compiler_params=pltpu.CompilerParams(needs_layout_passes=False),
    )(_unsort_body)
    return f(sti_flat, sow_flat)


def _comb_body(o_ref, wout_ref, bout_ref, out_ref):
    lg = o_ref[:, :, :, D:D + 1]
    m = jnp.max(lg, axis=1, keepdims=True)
    lse = m + jnp.log(jnp.sum(jnp.exp(lg - m), axis=1, keepdims=True))
    w = jnp.exp(lg - lse)
    acc = jnp.zeros((HEADS, 256, D), jnp.float32)
    for hh in range(N_HASHES):
        acc = acc + o_ref[:, hh, :, 0:D] * w[:, hh]
    cat = jnp.concatenate([acc[h] for h in range(HEADS)], axis=1)
    out = jnp.dot(cat, wout_ref[...], preferred_element_type=jnp.float32)
    out_ref[0] = out + bout_ref[...]


def _stage5(ouw4, Wout, bout2):
    tiles = N // 256
    return pl.pallas_call(
        _comb_body,
        grid=(B * tiles,),
        in_specs=[
            pl.BlockSpec((HEADS, N_HASHES, 256, 2 * D),
                         lambda i: (i // tiles, 0, i % tiles, 0)),
            pl.BlockSpec((C, C), lambda i: (0, 0)),
            pl.BlockSpec((1, C), lambda i: (0, 0)),
        ],
        out_specs=pl.BlockSpec((1, 256, C), lambda i: (i // tiles, i % tiles, 0)),
        out_shape=jax.ShapeDtypeStruct((B, N, C), jnp.float32),
    )(ouw4, Wout, bout2)


def kernel(queries, keys, values, attn_mask, tau, delta, Wqk, Wv, Wout, bout,
           rotations):
    rotf = rotations.reshape(D, N_HASHES, 16)
    rot2 = jnp.transpose(
        jnp.concatenate([rotf, -rotf], axis=2), (0, 2, 1)).reshape(D, 128)
    qkv, bkt = _stage1(queries, Wqk, Wv, rot2)

    sti, stf, sqkv = _stage2(bkt.reshape(ROWS * N), qkv.reshape(BH * N, 2 * D))

    (sow,) = _stage3(
        sqkv.reshape(BH, N_HASHES * N, 2 * D),
        stf.reshape(BH, NCH, BUCKET_SIZE))

    (ouw,) = _stage4(sti, sow.reshape(ROWS * N, 2 * D))

    ouw4 = ouw.reshape(BH, N_HASHES, N, 2 * D)
    out = _stage5(ouw4, Wout, bout.reshape(1, C))
    return out


# speedup vs baseline: 1.2776x; 1.2776x over previous
import functools

import jax
import jax.numpy as jnp
from jax import lax
from jax.experimental import pallas as pl
from jax.experimental.pallas import tpu as pltpu
from jax.experimental.pallas import tpu_sc as plsc

HEADS = 16
BUCKET_SIZE = 64
N_HASHES = 4
TOKEN_SELF_ATTN_VALUE = -5e4

B, N, C = 2, 2048, 1024
D = C // HEADS
BH = B * HEADS
NB = N // BUCKET_SIZE
NCH = N_HASHES * NB
ROWS = BH * N_HASHES
NW = 32
RPW = ROWS // NW
GCH = 128
NG = N // GCH


def _proj_body(q_ref, wqk_ref, wv_ref, rot_ref, qkv_ref, bkt_ref):
    x = q_ref[0]
    qk = jnp.dot(x, wqk_ref[...], preferred_element_type=jnp.float32)
    v = jnp.dot(x, wv_ref[...], preferred_element_type=jnp.float32)
    rot = rot_ref[...]
    c = lax.broadcasted_iota(jnp.int32, (128, 4), 0)
    hcol = lax.broadcasted_iota(jnp.int32, (128, 4), 1)
    pw = lax.bitcast_convert_type((158 - c // 4) << 23, jnp.float32)
    selw = jnp.where((c % 4) == hcol, pw, 0.0)
    for h in range(HEADS):
        qh = qk[:, h * D:(h + 1) * D]
        qkv_ref[h] = jnp.concatenate([qh, v[:, h * D:(h + 1) * D]], axis=1)
        xf = jnp.dot(qh, rot, preferred_element_type=jnp.float32)
        t = jnp.maximum(xf[:, 0:64], xf[:, 64:128])
        t = jnp.maximum(t[:, 0:32], t[:, 32:64])
        t = jnp.maximum(t[:, 0:16], t[:, 16:32])
        t = jnp.maximum(t[:, 0:8], t[:, 8:16])
        mb = jnp.maximum(t[:, 0:4], t[:, 4:8])
        for _ in range(5):
            mb = jnp.concatenate([mb, mb], axis=1)
        eq = jnp.where(xf == mb, 1.0, 0.0)
        sc_ = jnp.dot(eq, selw, preferred_element_type=jnp.float32)
        scT = jnp.transpose(sc_, (1, 0))
        bkt_ref[h] = 158 - (lax.bitcast_convert_type(scT, jnp.int32) >> 23)


def _stage1(queries, Wqk, Wv, rot2):
    tiles = N // 256
    grid = (B * tiles,)
    return pl.pallas_call(
        _proj_body,
        grid=grid,
        in_specs=[
            pl.BlockSpec((1, 256, C), lambda i: (i // tiles, i % tiles, 0)),
            pl.BlockSpec((C, C), lambda i: (0, 0)),
            pl.BlockSpec((C, C), lambda i: (0, 0)),
            pl.BlockSpec((D, 128), lambda i: (0, 0)),
        ],
        out_specs=[
            pl.BlockSpec((HEADS, 256, 2 * D), lambda i: (i // tiles, i % tiles, 0)),
            pl.BlockSpec((HEADS, N_HASHES, 256), lambda i: (i // tiles, 0, i % tiles)),
        ],
        out_shape=[
            jax.ShapeDtypeStruct((BH, N, 2 * D), jnp.float32),
            jax.ShapeDtypeStruct((BH, N_HASHES, N), jnp.int32),
        ],
    )(queries, Wqk, Wv, rot2)


def _sort_gather_body(bkt_hbm, qkv_hbm,
                      sti_hbm, stf_hbm, sqkv_hbm,
                      bktv, stv, stfv, gidxv, cntv, curv, gbuf, sem):
    cc = lax.axis_index("c")
    ss = lax.axis_index("s")
    wid = ss * 2 + cc
    zeros16 = jnp.zeros((16,), jnp.int32)
    for rr in range(RPW):
        row = wid * RPW + rr
        bh = row // N_HASHES
        base = row * N
        pltpu.sync_copy(bkt_hbm.at[pl.ds(base, N)], bktv)
        cntv[pl.ds(0, 16)] = zeros16
        cntv[pl.ds(16, 16)] = zeros16

        def p1(t, carry):
            vec = bktv[pl.ds(t * 16, 16)]
            sc, lastm = plsc.scan_count(vec)
            plsc.addupdate_scatter(cntv, [vec], sc, mask=lastm)
            return carry

        lax.fori_loop(0, N // 16, p1, 0)

        c0 = cntv[pl.ds(0, 16)]
        c1 = cntv[pl.ds(16, 16)]
        ic0 = plsc.cumsum(c0)
        ic1 = plsc.cumsum(c1)
        tot0 = jnp.sum(c0)
        curv[pl.ds(0, 16)] = ic0 - c0
        curv[pl.ds(16, 16)] = ic1 - c1 + tot0

        gbase = bh * N

        def p2(t, carry):
            vec = bktv[pl.ds(t * 16, 16)]
            sc, lastm = plsc.scan_count(vec)
            bse = plsc.load_gather(curv, [vec])
            dest = bse + sc - 1
            tok = t * 16 + lax.iota(jnp.int32, 16)
            plsc.store_scatter(stv, [dest], tok)
            plsc.store_scatter(stfv, [dest], tok.astype(jnp.float32))
            plsc.store_scatter(gidxv, [dest], tok + gbase)
            plsc.store_scatter(curv, [vec], bse + sc, mask=lastm)
            return carry

        lax.fori_loop(0, N // 16, p2, 0)

        pltpu.sync_copy(stv, sti_hbm.at[pl.ds(base, N)])
        pltpu.sync_copy(stfv, stf_hbm.at[pl.ds(base, N)])
        for k in range(NG):
            idx = gidxv.at[pl.ds(k * GCH, GCH)]
            pltpu.async_copy(qkv_hbm.at[idx], gbuf, sem).wait()
            pltpu.sync_copy(gbuf, sqkv_hbm.at[pl.ds(base + k * GCH, GCH)])


def _stage2(bkt_flat, qkv_flat):
    mesh = plsc.VectorSubcoreMesh(core_axis_name="c", subcore_axis_name="s")
    f = functools.partial(
        pl.kernel,
        mesh=mesh,
        out_type=[
            jax.ShapeDtypeStruct((ROWS * N,), jnp.int32),
            jax.ShapeDtypeStruct((ROWS * N,), jnp.float32),
            jax.ShapeDtypeStruct((ROWS * N, 2 * D), jnp.float32),
        ],
        scratch_types=[
            pltpu.VMEM((N,), jnp.int32),
            pltpu.VMEM((N,), jnp.int32),
            pltpu.VMEM((N,), jnp.float32),
            pltpu.VMEM((N,), jnp.int32),
            pltpu.VMEM((NB,), jnp.int32),
            pltpu.VMEM((NB,), jnp.int32),
            pltpu.VMEM((GCH, 2 * D), jnp.float32),
            pltpu.SemaphoreType.DMA,
        ],
        compiler_params=pltpu.CompilerParams(needs_layout_passes=False),
    )(_sort_gather_body)
    return f(bkt_flat, qkv_flat)


ATT_G = 4


def _attn_body(sqkv_ref, stf_ref, sow_ref):
    G = ATT_G
    M = G * BUCKET_SIZE
    W = (G + 1) * BUCKET_SIZE
    ii = lax.broadcasted_iota(jnp.int32, (M, W), 0)
    jj = lax.broadcasted_iota(jnp.int32, (M, W), 1)
    qc = ii // BUCKET_SIZE
    kc = jj // BUCKET_SIZE
    band = jnp.logical_or(kc == qc, kc == qc + 1)
    selfd = jj == ii + BUCKET_SIZE
    repl_mask = jnp.logical_or(selfd, jnp.logical_not(band))
    repl_val = jnp.where(selfd, TOKEN_SELF_ATTN_VALUE, -1e30)
    for g in range(NCH // G):
        c0 = g * G
        pc = (c0 - 1) % NCH
        qall = sqkv_ref[0, c0 * BUCKET_SIZE:c0 * BUCKET_SIZE + M, :]
        q = qall[:, 0:D]
        vcur = qall[:, D:2 * D]
        pall = sqkv_ref[0, pc * BUCKET_SIZE:(pc + 1) * BUCKET_SIZE, :]
        kcat = jnp.concatenate([pall[:, 0:D], q], axis=0)
        nrm = jnp.sqrt(jnp.sum(kcat * kcat, axis=1, keepdims=True))
        kn = kcat / (jnp.maximum(nrm, 1e-12) * (D ** 0.5))
        dots = lax.dot_general(
            q.astype(jnp.bfloat16), kn.astype(jnp.bfloat16),
            (((1,), (1,)), ((), ())),
            preferred_element_type=jnp.float32)
        dots = jnp.where(repl_mask, repl_val, dots)
        if c0 % NB == 0:
            qid = jnp.reshape(stf_ref[0, pl.ds(c0, 1), :],
                              (BUCKET_SIZE, 1))
            pid = stf_ref[0, pl.ds(pc, 1), :]
            eq = jnp.where(qid == pid, 1.0, 0.0)
            eq = jnp.concatenate(
                [eq, jnp.zeros((BUCKET_SIZE, W - BUCKET_SIZE), jnp.float32)],
                axis=1)
            eq = jnp.concatenate(
                [eq, jnp.zeros((M - BUCKET_SIZE, W), jnp.float32)], axis=0)
            dots = jnp.where(eq > 0.5, TOKEN_SELF_ATTN_VALUE, dots)
        m = jnp.max(dots, axis=1, keepdims=True)
        ex = jnp.exp(dots - m)
        ssum = jnp.sum(ex, axis=1, keepdims=True)
        lse = m + jnp.log(ssum)
        p = ex * (1.0 / ssum)
        vcat = jnp.concatenate([pall[:, D:2 * D], vcur], axis=0)
        bo = lax.dot_general(
            p.astype(jnp.bfloat16), vcat.astype(jnp.bfloat16),
            (((1,), (0,)), ((), ())),
            preferred_element_type=jnp.float32)
        packed = jnp.concatenate(
            [bo, jnp.broadcast_to(lse, (M, D))], axis=1)
        sow_ref[0, c0 * BUCKET_SIZE:c0 * BUCKET_SIZE + M, :] = packed


def _stage3(sqkv3, stf3):
    return pl.pallas_call(
        _attn_body,
        grid=(BH,),
        in_specs=[
            pl.BlockSpec((1, N_HASHES * N, 2 * D), lambda i: (i, 0, 0)),
            pl.BlockSpec((1, NCH, BUCKET_SIZE), lambda i: (i, 0, 0)),
        ],
        out_specs=[
            pl.BlockSpec((1, N_HASHES * N, 2 * D), lambda i: (i, 0, 0)),
        ],
        out_shape=[
            jax.ShapeDtypeStruct((BH, N_HASHES * N, 2 * D), jnp.float32),
        ],
    )(sqkv3, stf3)


def _unsort_body(sti_hbm, sow_hbm, ouw_hbm, stv, sidx, obuf, sem):
    cc = lax.axis_index("c")
    ss = lax.axis_index("s")
    wid = ss * 2 + cc
    for rr in range(RPW):
        row = wid * RPW + rr
        base = row * N
        pltpu.sync_copy(sti_hbm.at[pl.ds(base, N)], stv)

        for k in range(NG):
            def sk(j, carry):
                sidx[k, pl.ds(j * 16, 16)] = (
                    stv[pl.ds(k * GCH + j * 16, 16)] + base)
                return carry

            lax.fori_loop(0, GCH // 16, sk, 0)

        for k in range(NG):
            pltpu.sync_copy(sow_hbm.at[pl.ds(base + k * GCH, GCH)], obuf)
            pltpu.async_copy(obuf, ouw_hbm.at[sidx.at[k]], sem).wait()


def _stage4(sti_flat, sow_flat):
    mesh = plsc.VectorSubcoreMesh(core_axis_name="c", subcore_axis_name="s")
    f = functools.partial(
        pl.kernel,
        mesh=mesh,
        out_type=[
            jax.ShapeDtypeStruct((ROWS * N, 2 * D), jnp.float32),
        ],
        scratch_types=[
            pltpu.VMEM((N,), jnp.int32),
            pltpu.VMEM((NG, GCH), jnp.int32),
            pltpu.VMEM((GCH, 2 * D), jnp.float32),
            pltpu.SemaphoreType.DMA,
        ],
        compiler_params=pltpu.CompilerParams(needs_layout_passes=False),
    )(_unsort_body)
    return f(sti_flat, sow_flat)


def _comb_body(o_ref, wout_ref, bout_ref, out_ref):
    lg = o_ref[:, :, :, D:D + 1]
    m = jnp.max(lg, axis=1, keepdims=True)
    lse = m + jnp.log(jnp.sum(jnp.exp(lg - m), axis=1, keepdims=True))
    w = jnp.exp(lg - lse)
    acc = jnp.zeros((HEADS, 256, D), jnp.float32)
    for hh in range(N_HASHES):
        acc = acc + o_ref[:, hh, :, 0:D] * w[:, hh]
    cat = jnp.concatenate([acc[h] for h in range(HEADS)], axis=1)
    out = jnp.dot(cat, wout_ref[...], preferred_element_type=jnp.float32)
    out_ref[0] = out + bout_ref[...]


def _stage5(ouw4, Wout, bout2):
    tiles = N // 256
    return pl.pallas_call(
        _comb_body,
        grid=(B * tiles,),
        in_specs=[
            pl.BlockSpec((HEADS, N_HASHES, 256, 2 * D),
                         lambda i: (i // tiles, 0, i % tiles, 0)),
            pl.BlockSpec((C, C), lambda i: (0, 0)),
            pl.BlockSpec((1, C), lambda i: (0, 0)),
        ],
        out_specs=pl.BlockSpec((1, 256, C), lambda i: (i // tiles, i % tiles, 0)),
        out_shape=jax.ShapeDtypeStruct((B, N, C), jnp.float32),
    )(ouw4, Wout, bout2)


def kernel(queries, keys, values, attn_mask, tau, delta, Wqk, Wv, Wout, bout,
           rotations):
    rotf = rotations.reshape(D, N_HASHES, 16)
    rot2 = jnp.transpose(
        jnp.concatenate([rotf, -rotf], axis=2), (0, 2, 1)).reshape(D, 128)
    qkv, bkt = _stage1(queries, Wqk, Wv, rot2)

    sti, stf, sqkv = _stage2(bkt.reshape(ROWS * N), qkv.reshape(BH * N, 2 * D))

    (sow,) = _stage3(
        sqkv.reshape(BH, N_HASHES * N, 2 * D),
        stf.reshape(BH, NCH, BUCKET_SIZE))

    (ouw,) = _stage4(sti, sow.reshape(ROWS * N, 2 * D))

    ouw4 = ouw.reshape(BH, N_HASHES, N, 2 * D)
    out = _stage5(ouw4, Wout, bout.reshape(1, C))
    return out
